# Initial kernel scaffold; baseline (speedup 1.0000x reference)
import jax
import jax.numpy as jnp
from jax.experimental import pallas as pl


def kernel(h, e_src, e_dst, e_attr3, li_ws, li_bs, li_wv, rad_w1, rad_w2, res_ws, res_bs, res_wv):
    raise NotImplementedError("write your pallas kernel here")


# speedup vs baseline: 2.6532x; 2.6532x over previous
import functools

import jax
import jax.numpy as jnp
import numpy as np
from jax import lax
from jax.experimental import pallas as pl
from jax.experimental.pallas import tpu as pltpu
from jax.experimental.pallas import tpu_sc as plsc

_MUL = 32
_DIM = 128
_RH = 32
_SQ3 = float(np.sqrt(3.0))
_SCALE = float(1.0 / np.sqrt(2.0 * _MUL) / np.sqrt(_RH))

_NC = 2
_NS = 16


def _perm_planar() -> np.ndarray:
    perm = np.zeros(_DIM, dtype=np.int64)
    perm[:_MUL] = np.arange(_MUL)
    for m in range(3):
        for i in range(_MUL):
            perm[_MUL + _MUL * m + i] = _MUL + 3 * i + m
    return np.eye(_DIM, dtype=np.float32)[perm].T


_P_PLANAR = _perm_planar()
_P_OUT = np.ascontiguousarray(_P_PLANAR.T)


def _equiv_weight(ws, wv):
    bd = jnp.zeros((_DIM, _DIM), jnp.float32)
    bd = bd.at[:_MUL, :_MUL].set(ws / np.sqrt(_MUL))
    w = wv / np.sqrt(_MUL)
    for m in range(3):
        a = _MUL + _MUL * m
        bd = bd.at[a:a + _MUL, a:a + _MUL].set(w)
    return jnp.dot(jnp.asarray(_P_PLANAR), bd)


def _pre_body(h_ref, w_ref, b_ref, esrc_ref, edst_ref,
              hin_ref, fsrc_ref, fdst_ref):
    hin_ref[...] = jnp.dot(h_ref[...], w_ref[...],
                           preferred_element_type=jnp.float32) + b_ref[...]
    n = hin_ref.shape[0] // esrc_ref.shape[0]
    boff = lax.broadcasted_iota(jnp.int32, esrc_ref.shape, 0) * n
    fsrc_ref[...] = esrc_ref[...] + boff
    fdst_ref[...] = edst_ref[...] + boff


def _pre_call(h2, w_a, b_a, e_src, e_dst):
    bn = h2.shape[0]
    b, e = e_src.shape
    return pl.pallas_call(
        _pre_body,
        out_shape=[
            jax.ShapeDtypeStruct((bn, _DIM), jnp.float32),
            jax.ShapeDtypeStruct((b, e), jnp.int32),
            jax.ShapeDtypeStruct((b, e), jnp.int32),
        ],
    )(h2, w_a, b_a, e_src, e_dst)


def _gather_call(fsrc2, hin):
    edges = fsrc2.shape[0] * fsrc2.shape[1]
    per_w = edges // (_NC * _NS)
    rows_per_w = fsrc2.shape[1]
    k = per_w // rows_per_w
    mesh = plsc.VectorSubcoreMesh(core_axis_name="c", subcore_axis_name="s",
                                  num_cores=_NC, num_subcores=_NS)

    @functools.partial(
        pl.kernel,
        out_type=jax.ShapeDtypeStruct((edges, _DIM), jnp.float32),
        mesh=mesh,
        scratch_types=[
            pltpu.VMEM((k, rows_per_w), jnp.int32),
            pltpu.VMEM((per_w, _DIM), jnp.float32),
            pltpu.SemaphoreType.DMA,
        ],
    )
    def _gather(idx_hbm, table_hbm, out_hbm, idx_v, rows_v, sem):
        wid = lax.axis_index("s") * _NC + lax.axis_index("c")
        pltpu.sync_copy(idx_hbm.at[pl.ds(wid * k, k)], idx_v)
        cps = [
            pltpu.async_copy(table_hbm.at[idx_v.at[j]],
                             rows_v.at[pl.ds(j * rows_per_w, rows_per_w)], sem)
            for j in range(k)
        ]
        for c in cps:
            c.wait()
        pltpu.sync_copy(rows_v, out_hbm.at[pl.ds(wid * per_w, per_w)])

    return _gather(fsrc2, hin)


_EBLK = 128


def _msg_body(attr_ref, hsrc_ref, rw1_ref, ws_ref, w4_ref, w3_ref, rrep_ref,
              out_ref):
    att = attr_ref[...]
    r0, r1, r2 = att[:, 0:1], att[:, 1:2], att[:, 2:3]
    rnorm = jnp.sqrt(r0 * r0 + r1 * r1 + r2 * r2) + 1e-8
    inv = 1.0 / rnorm
    y0, y1, y2 = _SQ3 * r0 * inv, _SQ3 * r1 * inv, _SQ3 * r2 * inv
    pre = rnorm * rw1_ref[...]
    hid = pre * jax.nn.sigmoid(pre)
    hs = hsrc_ref[...]
    s = hs[:, 0:_MUL]
    v0 = hs[:, _MUL:2 * _MUL]
    v1 = hs[:, 2 * _MUL:3 * _MUL]
    v2 = hs[:, 3 * _MUL:4 * _MUL]
    inner = (v0 * y0 + v1 * y1 + v2 * y2) * (1.0 / _SQ3)
    hrep = jnp.dot(hid, rrep_ref[...], preferred_element_type=jnp.float32)

    def contract(t):
        x = hrep * t
        x = x[:, :512] + x[:, 512:]
        x = x[:, :256] + x[:, 256:]
        x = x[:, :128] + x[:, 128:]
        x = x[:, :64] + x[:, 64:]
        return x[:, :32] + x[:, 32:]

    ts = jnp.dot(s, ws_ref[...], preferred_element_type=jnp.float32)
    o1 = contract(ts[:, :1024])
    o2 = contract(ts[:, 1024:])
    o4 = contract(jnp.dot(inner, w4_ref[...],
                          preferred_element_type=jnp.float32))
    o30 = contract(jnp.dot(v0, w3_ref[...],
                           preferred_element_type=jnp.float32))
    o31 = contract(jnp.dot(v1, w3_ref[...],
                           preferred_element_type=jnp.float32))
    o32 = contract(jnp.dot(v2, w3_ref[...],
                           preferred_element_type=jnp.float32))
    m_s = _SCALE * (o1 + o4)
    out_ref[...] = jnp.concatenate(
        [m_s,
         _SCALE * (o2 * y0 + o30),
         _SCALE * (o2 * y1 + o31),
         _SCALE * (o2 * y2 + o32)], axis=1)


def _msg_call(attr2, hsrc, rad_w1, w_s, w_4, w_3, r_rep):
    edges = attr2.shape[0]
    grid = edges // _EBLK
    return pl.pallas_call(
        _msg_body,
        grid=(grid,),
        in_specs=[
            pl.BlockSpec((_EBLK, 3), lambda i: (i, 0)),
            pl.BlockSpec((_EBLK, _DIM), lambda i: (i, 0)),
            pl.BlockSpec((1, _RH), lambda i: (0, 0)),
            pl.BlockSpec((_RH, 2048), lambda i: (0, 0)),
            pl.BlockSpec((_RH, 1024), lambda i: (0, 0)),
            pl.BlockSpec((_RH, 1024), lambda i: (0, 0)),
            pl.BlockSpec((_RH, 1024), lambda i: (0, 0)),
        ],
        out_specs=pl.BlockSpec((_EBLK, _DIM), lambda i: (i, 0)),
        out_shape=jax.ShapeDtypeStruct((edges, _DIM), jnp.float32),
    )(attr2, hsrc, rad_w1, w_s, w_4, w_3, r_rep)


def _scatter_call(fdst2, msg, zeros_hbm):
    edges = fdst2.shape[0] * fdst2.shape[1]
    bn = zeros_hbm.shape[0]
    per_w = edges // (_NC * _NS)
    rows_per_w = fdst2.shape[1]
    k = per_w // rows_per_w
    rows_per_tile = bn // _NS
    mesh = plsc.VectorSubcoreMesh(core_axis_name="c", subcore_axis_name="s",
                                  num_cores=_NC, num_subcores=_NS)

    @functools.partial(
        pl.kernel,
        out_type=jax.ShapeDtypeStruct((_NC, bn, _DIM), jnp.float32),
        mesh=mesh,
        scratch_types=[
            pltpu.VMEM((k, rows_per_w), jnp.int32),
            pltpu.VMEM((per_w, _DIM), jnp.float32),
            pltpu.VMEM_SHARED((bn, _DIM), jnp.float32),
        ],
    )
    def _scatter(idx_hbm, msg_hbm, z_hbm, out_hbm, idx_v, rows_v, acc):
        cid = lax.axis_index("c")
        sid = lax.axis_index("s")
        wid = sid * _NC + cid
        r0 = sid * rows_per_tile
        pltpu.sync_copy(z_hbm.at[pl.ds(r0, rows_per_tile)],
                        acc.at[pl.ds(r0, rows_per_tile)])
        pltpu.sync_copy(idx_hbm.at[pl.ds(wid * k, k)], idx_v)
        pltpu.sync_copy(msg_hbm.at[pl.ds(wid * per_w, per_w)], rows_v)
        plsc.subcore_barrier()
        for j in range(k):
            pltpu.sync_copy(rows_v.at[pl.ds(j * rows_per_w, rows_per_w)],
                            acc.at[idx_v.at[j]], add=True)
        plsc.subcore_barrier()
        pltpu.sync_copy(acc.at[pl.ds(r0, rows_per_tile)],
                        out_hbm.at[cid, pl.ds(r0, rows_per_tile)])

    return _scatter(fdst2, msg, zeros_hbm)


def _fin_body(part_ref, h_ref, wr_ref, br_ref, pout_ref, out_ref):
    res = jnp.dot(h_ref[...], wr_ref[...],
                  preferred_element_type=jnp.float32) + br_ref[...]
    x = part_ref[0] + part_ref[1] + res
    s = x[:, 0:_MUL]
    v0 = x[:, _MUL:2 * _MUL]
    v1 = x[:, 2 * _MUL:3 * _MUL]
    v2 = x[:, 3 * _MUL:4 * _MUL]
    ns = jnp.abs(s)
    s2 = s * (ns * jax.nn.sigmoid(ns)) / (ns + 1e-8)
    nv = jnp.sqrt(v0 * v0 + v1 * v1 + v2 * v2)
    f = (nv * jax.nn.sigmoid(nv)) / (nv + 1e-8)
    y = jnp.concatenate([s2, v0 * f, v1 * f, v2 * f], axis=1)
    out_ref[...] = jnp.dot(y, pout_ref[...],
                           preferred_element_type=jnp.float32)


def _fin_call(parts, h2, w_r, b_r, pout):
    bn = h2.shape[0]
    return pl.pallas_call(
        _fin_body,
        out_shape=jax.ShapeDtypeStruct((bn, _DIM), jnp.float32),
    )(parts, h2, w_r, b_r, pout)


def kernel(h, e_src, e_dst, e_attr3, li_ws, li_bs, li_wv,
           rad_w1, rad_w2, res_ws, res_bs, res_wv):
    b, n, c = h.shape
    e = e_attr3.shape[1]
    edges = b * e

    w_a = _equiv_weight(li_ws, li_wv)
    b_a = jnp.concatenate([li_bs, jnp.zeros((3 * _MUL,), jnp.float32)])[None]
    w_r = _equiv_weight(res_ws, res_wv)
    b_r = jnp.concatenate([res_bs, jnp.zeros((3 * _MUL,), jnp.float32)])[None]
    rad4 = rad_w2.reshape(_RH, 4, _MUL, _MUL)
    def perm(p):
        return rad4[:, p].transpose(1, 0, 2).reshape(_MUL, _RH * _MUL)
    w_s = jnp.concatenate([perm(0), perm(1)], axis=1)
    w_4 = perm(3)
    w_3 = perm(2)
    r_rep = jnp.asarray(np.repeat(np.eye(_RH, dtype=np.float32), _MUL, axis=1))
    pout = jnp.asarray(_P_OUT)
    zeros = jnp.zeros((b * n, _DIM), jnp.float32)

    h2 = h.reshape(b * n, c)
    hin, fsrc, fdst = _pre_call(h2, w_a, b_a, e_src, e_dst)
    idx_cols = 128
    hsrc = _gather_call(fsrc.reshape(edges // idx_cols, idx_cols), hin)
    msg = _msg_call(e_attr3.reshape(edges, 3), hsrc,
                    rad_w1, w_s, w_4, w_3, r_rep)
    parts = _scatter_call(fdst.reshape(edges // idx_cols, idx_cols), msg, zeros)
    out = _fin_call(parts, h2, w_r, b_r, pout)
    return out.reshape(b, n, c)


# speedup vs baseline: 3.2089x; 1.2095x over previous
import functools

import jax
import jax.numpy as jnp
import numpy as np
from jax import lax
from jax.experimental import pallas as pl
from jax.experimental.pallas import tpu as pltpu
from jax.experimental.pallas import tpu_sc as plsc

_MUL = 32
_DIM = 128
_RH = 32
_SQ3 = float(np.sqrt(3.0))
_SCALE = float(1.0 / np.sqrt(2.0 * _MUL) / np.sqrt(_RH))

_NC = 2
_NS = 16


def _perm_planar() -> np.ndarray:
    perm = np.zeros(_DIM, dtype=np.int64)
    perm[:_MUL] = np.arange(_MUL)
    for m in range(3):
        for i in range(_MUL):
            perm[_MUL + _MUL * m + i] = _MUL + 3 * i + m
    return np.eye(_DIM, dtype=np.float32)[perm].T


_P_PLANAR = _perm_planar()
_P_OUT = np.ascontiguousarray(_P_PLANAR.T)


def _equiv_weight(ws, wv):
    bd = jnp.zeros((_DIM, _DIM), jnp.float32)
    bd = bd.at[:_MUL, :_MUL].set(ws / np.sqrt(_MUL))
    w = wv / np.sqrt(_MUL)
    for m in range(3):
        a = _MUL + _MUL * m
        bd = bd.at[a:a + _MUL, a:a + _MUL].set(w)
    return jnp.dot(jnp.asarray(_P_PLANAR), bd)


def _pre_body(h_ref, w_ref, b_ref, esrc_ref, edst_ref,
              hin_ref, fsrc_ref, fdst_ref):
    hin_ref[...] = jnp.dot(h_ref[...], w_ref[...],
                           preferred_element_type=jnp.float32) + b_ref[...]
    n = hin_ref.shape[0] // esrc_ref.shape[0]
    boff = lax.broadcasted_iota(jnp.int32, esrc_ref.shape, 0) * n
    fsrc_ref[...] = esrc_ref[...] + boff
    fdst_ref[...] = edst_ref[...] + boff


def _pre_call(h2, w_a, b_a, e_src, e_dst):
    bn = h2.shape[0]
    b, e = e_src.shape
    return pl.pallas_call(
        _pre_body,
        out_shape=[
            jax.ShapeDtypeStruct((bn, _DIM), jnp.float32),
            jax.ShapeDtypeStruct((b, e), jnp.int32),
            jax.ShapeDtypeStruct((b, e), jnp.int32),
        ],
    )(h2, w_a, b_a, e_src, e_dst)


def _gather_call(fsrc2, hin):
    edges = fsrc2.shape[0] * fsrc2.shape[1]
    per_w = edges // (_NC * _NS)
    rows_per_w = fsrc2.shape[1]
    k = per_w // rows_per_w
    mesh = plsc.VectorSubcoreMesh(core_axis_name="c", subcore_axis_name="s",
                                  num_cores=_NC, num_subcores=_NS)

    @functools.partial(
        pl.kernel,
        out_type=jax.ShapeDtypeStruct((edges, _DIM), jnp.float32),
        mesh=mesh,
        scratch_types=[
            pltpu.VMEM((k, rows_per_w), jnp.int32),
            pltpu.VMEM((per_w, _DIM), jnp.float32),
            pltpu.SemaphoreType.DMA,
        ],
    )
    def _gather(idx_hbm, table_hbm, out_hbm, idx_v, rows_v, sem):
        wid = lax.axis_index("s") * _NC + lax.axis_index("c")
        pltpu.sync_copy(idx_hbm.at[pl.ds(wid * k, k)], idx_v)
        cps = [
            pltpu.async_copy(table_hbm.at[idx_v.at[j]],
                             rows_v.at[pl.ds(j * rows_per_w, rows_per_w)], sem)
            for j in range(k)
        ]
        for c in cps:
            c.wait()
        pltpu.sync_copy(rows_v, out_hbm.at[pl.ds(wid * per_w, per_w)])

    return _gather(fsrc2, hin)


_EBLK = 256


def _msg_body(attr_ref, hsrc_ref, rw1_ref, ws_ref, w4_ref, w3_ref, rrep_ref,
              out_ref):
    att = attr_ref[...]
    r0, r1, r2 = att[:, 0:1], att[:, 1:2], att[:, 2:3]
    rnorm = jnp.sqrt(r0 * r0 + r1 * r1 + r2 * r2) + 1e-8
    inv = 1.0 / rnorm
    y0, y1, y2 = _SQ3 * r0 * inv, _SQ3 * r1 * inv, _SQ3 * r2 * inv
    pre = rnorm * rw1_ref[...]
    hid = pre * jax.nn.sigmoid(pre)
    hs = hsrc_ref[...]
    s = hs[:, 0:_MUL]
    v0 = hs[:, _MUL:2 * _MUL]
    v1 = hs[:, 2 * _MUL:3 * _MUL]
    v2 = hs[:, 3 * _MUL:4 * _MUL]
    inner = (v0 * y0 + v1 * y1 + v2 * y2) * (1.0 / _SQ3)
    hrep = jnp.dot(hid, rrep_ref[...], preferred_element_type=jnp.float32)
    hlo, hhi = hrep[:, :512], hrep[:, 512:]
    bf = jnp.bfloat16

    def contract(t):
        x = hlo * t[:, :512] + hhi * t[:, 512:]
        x = x[:, :256] + x[:, 256:]
        x = x[:, :128] + x[:, 128:]
        x = x[:, :64] + x[:, 64:]
        return x[:, :32] + x[:, 32:]

    def bdot(a, w):
        return jnp.dot(a.astype(bf), w.astype(bf),
                       preferred_element_type=jnp.float32)

    ts = bdot(s, ws_ref[...])
    o1 = contract(ts[:, :1024])
    o2 = contract(ts[:, 1024:])
    o4 = contract(bdot(inner, w4_ref[...]))
    o30 = contract(bdot(v0, w3_ref[...]))
    o31 = contract(bdot(v1, w3_ref[...]))
    o32 = contract(bdot(v2, w3_ref[...]))
    m_s = _SCALE * (o1 + o4)
    out_ref[...] = jnp.concatenate(
        [m_s,
         _SCALE * (o2 * y0 + o30),
         _SCALE * (o2 * y1 + o31),
         _SCALE * (o2 * y2 + o32)], axis=1)


def _msg_call(attr2, hsrc, rad_w1, w_s, w_4, w_3, r_rep):
    edges = attr2.shape[0]
    grid = edges // _EBLK
    return pl.pallas_call(
        _msg_body,
        grid=(grid,),
        in_specs=[
            pl.BlockSpec((_EBLK, 3), lambda i: (i, 0)),
            pl.BlockSpec((_EBLK, _DIM), lambda i: (i, 0)),
            pl.BlockSpec((1, _RH), lambda i: (0, 0)),
            pl.BlockSpec((_RH, 2048), lambda i: (0, 0)),
            pl.BlockSpec((_RH, 1024), lambda i: (0, 0)),
            pl.BlockSpec((_RH, 1024), lambda i: (0, 0)),
            pl.BlockSpec((_RH, 1024), lambda i: (0, 0)),
        ],
        out_specs=pl.BlockSpec((_EBLK, _DIM), lambda i: (i, 0)),
        out_shape=jax.ShapeDtypeStruct((edges, _DIM), jnp.float32),
    )(attr2, hsrc, rad_w1, w_s, w_4, w_3, r_rep)


def _scatter_call(fdst2, msg, zeros_hbm):
    edges = fdst2.shape[0] * fdst2.shape[1]
    bn = zeros_hbm.shape[0]
    per_w = edges // (_NC * _NS)
    rows_per_w = fdst2.shape[1]
    k = per_w // rows_per_w
    rows_per_tile = bn // _NS
    mesh = plsc.VectorSubcoreMesh(core_axis_name="c", subcore_axis_name="s",
                                  num_cores=_NC, num_subcores=_NS)

    @functools.partial(
        pl.kernel,
        out_type=jax.ShapeDtypeStruct((_NC, bn, _DIM), jnp.float32),
        mesh=mesh,
        scratch_types=[
            pltpu.VMEM((k, rows_per_w), jnp.int32),
            pltpu.VMEM((per_w, _DIM), jnp.float32),
            pltpu.VMEM_SHARED((bn, _DIM), jnp.float32),
        ],
    )
    def _scatter(idx_hbm, msg_hbm, z_hbm, out_hbm, idx_v, rows_v, acc):
        cid = lax.axis_index("c")
        sid = lax.axis_index("s")
        wid = sid * _NC + cid
        r0 = sid * rows_per_tile
        pltpu.sync_copy(z_hbm.at[pl.ds(r0, rows_per_tile)],
                        acc.at[pl.ds(r0, rows_per_tile)])
        pltpu.sync_copy(idx_hbm.at[pl.ds(wid * k, k)], idx_v)
        pltpu.sync_copy(msg_hbm.at[pl.ds(wid * per_w, per_w)], rows_v)
        plsc.subcore_barrier()
        for j in range(k):
            pltpu.sync_copy(rows_v.at[pl.ds(j * rows_per_w, rows_per_w)],
                            acc.at[idx_v.at[j]], add=True)
        plsc.subcore_barrier()
        pltpu.sync_copy(acc.at[pl.ds(r0, rows_per_tile)],
                        out_hbm.at[cid, pl.ds(r0, rows_per_tile)])

    return _scatter(fdst2, msg, zeros_hbm)


def _fin_body(part_ref, h_ref, wr_ref, br_ref, pout_ref, out_ref):
    res = jnp.dot(h_ref[...], wr_ref[...],
                  preferred_element_type=jnp.float32) + br_ref[...]
    x = part_ref[0] + part_ref[1] + res
    s = x[:, 0:_MUL]
    v0 = x[:, _MUL:2 * _MUL]
    v1 = x[:, 2 * _MUL:3 * _MUL]
    v2 = x[:, 3 * _MUL:4 * _MUL]
    ns = jnp.abs(s)
    s2 = s * (ns * jax.nn.sigmoid(ns)) / (ns + 1e-8)
    nv = jnp.sqrt(v0 * v0 + v1 * v1 + v2 * v2)
    f = (nv * jax.nn.sigmoid(nv)) / (nv + 1e-8)
    y = jnp.concatenate([s2, v0 * f, v1 * f, v2 * f], axis=1)
    out_ref[...] = jnp.dot(y, pout_ref[...],
                           preferred_element_type=jnp.float32)


def _fin_call(parts, h2, w_r, b_r, pout):
    bn = h2.shape[0]
    return pl.pallas_call(
        _fin_body,
        out_shape=jax.ShapeDtypeStruct((bn, _DIM), jnp.float32),
    )(parts, h2, w_r, b_r, pout)


def kernel(h, e_src, e_dst, e_attr3, li_ws, li_bs, li_wv,
           rad_w1, rad_w2, res_ws, res_bs, res_wv):
    b, n, c = h.shape
    e = e_attr3.shape[1]
    edges = b * e

    w_a = _equiv_weight(li_ws, li_wv)
    b_a = jnp.concatenate([li_bs, jnp.zeros((3 * _MUL,), jnp.float32)])[None]
    w_r = _equiv_weight(res_ws, res_wv)
    b_r = jnp.concatenate([res_bs, jnp.zeros((3 * _MUL,), jnp.float32)])[None]
    rad4 = rad_w2.reshape(_RH, 4, _MUL, _MUL)
    def perm(p):
        return rad4[:, p].transpose(1, 0, 2).reshape(_MUL, _RH * _MUL)
    w_s = jnp.concatenate([perm(0), perm(1)], axis=1)
    w_4 = perm(3)
    w_3 = perm(2)
    r_rep = jnp.asarray(np.repeat(np.eye(_RH, dtype=np.float32), _MUL, axis=1))
    pout = jnp.asarray(_P_OUT)
    zeros = jnp.zeros((b * n, _DIM), jnp.float32)

    h2 = h.reshape(b * n, c)
    hin, fsrc, fdst = _pre_call(h2, w_a, b_a, e_src, e_dst)
    idx_cols = 128
    hsrc = _gather_call(fsrc.reshape(edges // idx_cols, idx_cols), hin)
    msg = _msg_call(e_attr3.reshape(edges, 3), hsrc,
                    rad_w1, w_s, w_4, w_3, r_rep)
    parts = _scatter_call(fdst.reshape(edges // idx_cols, idx_cols), msg, zeros)
    out = _fin_call(parts, h2, w_r, b_r, pout)
    return out.reshape(b, n, c)


# speedup vs baseline: 3.2662x; 1.0178x over previous
import functools

import jax
import jax.numpy as jnp
import numpy as np
from jax import lax
from jax.experimental import pallas as pl
from jax.experimental.pallas import tpu as pltpu
from jax.experimental.pallas import tpu_sc as plsc

_MUL = 32
_DIM = 128
_RH = 32
_SQ3 = float(np.sqrt(3.0))
_SCALE = float(1.0 / np.sqrt(2.0 * _MUL) / np.sqrt(_RH))

_NC = 2
_NS = 16


def _perm_planar() -> np.ndarray:
    perm = np.zeros(_DIM, dtype=np.int64)
    perm[:_MUL] = np.arange(_MUL)
    for m in range(3):
        for i in range(_MUL):
            perm[_MUL + _MUL * m + i] = _MUL + 3 * i + m
    return np.eye(_DIM, dtype=np.float32)[perm].T


_P_PLANAR = _perm_planar()
_P_OUT = np.ascontiguousarray(_P_PLANAR.T)


def _equiv_planar(hp, ws, bs, wv):
    inv = float(1.0 / np.sqrt(_MUL))
    s = jnp.dot(hp[:, 0:_MUL], ws, preferred_element_type=jnp.float32) * inv
    s = s + bs
    outs = [s]
    for m in range(3):
        a = _MUL + _MUL * m
        outs.append(jnp.dot(hp[:, a:a + _MUL], wv,
                            preferred_element_type=jnp.float32) * inv)
    return jnp.concatenate(outs, axis=1)


def _pre_body(h_ref, p_ref, ws_ref, bs_ref, wv_ref, esrc_ref, edst_ref,
              hin_ref, fsrc_ref, fdst_ref):
    hp = jnp.dot(h_ref[...], p_ref[...], preferred_element_type=jnp.float32)
    hin_ref[...] = _equiv_planar(hp, ws_ref[...], bs_ref[...], wv_ref[...])
    n = hin_ref.shape[0] // esrc_ref.shape[0]
    boff = lax.broadcasted_iota(jnp.int32, esrc_ref.shape, 0) * n
    fsrc_ref[...] = esrc_ref[...] + boff
    fdst_ref[...] = edst_ref[...] + boff


def _pre_call(h2, p_in, li_ws, li_bs, li_wv, e_src, e_dst):
    bn = h2.shape[0]
    b, e = e_src.shape
    return pl.pallas_call(
        _pre_body,
        out_shape=[
            jax.ShapeDtypeStruct((bn, _DIM), jnp.float32),
            jax.ShapeDtypeStruct((b, e), jnp.int32),
            jax.ShapeDtypeStruct((b, e), jnp.int32),
        ],
    )(h2, p_in, li_ws, li_bs, li_wv, e_src, e_dst)


def _gather_call(fsrc2, hin):
    edges = fsrc2.shape[0] * fsrc2.shape[1]
    per_w = edges // (_NC * _NS)
    rows_per_w = fsrc2.shape[1]
    k = per_w // rows_per_w
    mesh = plsc.VectorSubcoreMesh(core_axis_name="c", subcore_axis_name="s",
                                  num_cores=_NC, num_subcores=_NS)

    @functools.partial(
        pl.kernel,
        out_type=jax.ShapeDtypeStruct((edges, _DIM), jnp.float32),
        mesh=mesh,
        scratch_types=[
            pltpu.VMEM((k, rows_per_w), jnp.int32),
            pltpu.VMEM((per_w, _DIM), jnp.float32),
            pltpu.SemaphoreType.DMA,
        ],
    )
    def _gather(idx_hbm, table_hbm, out_hbm, idx_v, rows_v, sem):
        wid = lax.axis_index("s") * _NC + lax.axis_index("c")
        pltpu.sync_copy(idx_hbm.at[pl.ds(wid * k, k)], idx_v)
        cps = [
            pltpu.async_copy(table_hbm.at[idx_v.at[j]],
                             rows_v.at[pl.ds(j * rows_per_w, rows_per_w)], sem)
            for j in range(k)
        ]
        for c in cps:
            c.wait()
        pltpu.sync_copy(rows_v, out_hbm.at[pl.ds(wid * per_w, per_w)])

    return _gather(fsrc2, hin)


_EBLK = 256


def _msg_body(attr_ref, hsrc_ref, rw1_ref, ws_ref, w4_ref, w3_ref, rrep_ref,
              out_ref):
    att = attr_ref[...]
    r0, r1, r2 = att[:, 0:1], att[:, 1:2], att[:, 2:3]
    rnorm = jnp.sqrt(r0 * r0 + r1 * r1 + r2 * r2) + 1e-8
    inv = 1.0 / rnorm
    y0, y1, y2 = _SQ3 * r0 * inv, _SQ3 * r1 * inv, _SQ3 * r2 * inv
    pre = rnorm * rw1_ref[...]
    hid = pre * jax.nn.sigmoid(pre)
    hs = hsrc_ref[...]
    s = hs[:, 0:_MUL]
    v0 = hs[:, _MUL:2 * _MUL]
    v1 = hs[:, 2 * _MUL:3 * _MUL]
    v2 = hs[:, 3 * _MUL:4 * _MUL]
    inner = (v0 * y0 + v1 * y1 + v2 * y2) * (1.0 / _SQ3)
    hrep = jnp.dot(hid, rrep_ref[...], preferred_element_type=jnp.float32)
    hlo, hhi = hrep[:, :512], hrep[:, 512:]
    bf = jnp.bfloat16

    def contract(t):
        x = hlo * t[:, :512] + hhi * t[:, 512:]
        x = x[:, :256] + x[:, 256:]
        x = x[:, :128] + x[:, 128:]
        x = x[:, :64] + x[:, 64:]
        return x[:, :32] + x[:, 32:]

    def bdot(a, w):
        return jnp.dot(a.astype(bf), w.astype(bf),
                       preferred_element_type=jnp.float32)

    ts = bdot(s, ws_ref[...])
    o1 = contract(ts[:, :1024])
    o2 = contract(ts[:, 1024:])
    o4 = contract(bdot(inner, w4_ref[...]))
    o30 = contract(bdot(v0, w3_ref[...]))
    o31 = contract(bdot(v1, w3_ref[...]))
    o32 = contract(bdot(v2, w3_ref[...]))
    m_s = _SCALE * (o1 + o4)
    out_ref[...] = jnp.concatenate(
        [m_s,
         _SCALE * (o2 * y0 + o30),
         _SCALE * (o2 * y1 + o31),
         _SCALE * (o2 * y2 + o32)], axis=1)


def _msg_call(attr2, hsrc, rad_w1, w_s, w_4, w_3, r_rep):
    edges = attr2.shape[0]
    grid = edges // _EBLK
    return pl.pallas_call(
        _msg_body,
        grid=(grid,),
        in_specs=[
            pl.BlockSpec((_EBLK, 3), lambda i: (i, 0)),
            pl.BlockSpec((_EBLK, _DIM), lambda i: (i, 0)),
            pl.BlockSpec((1, _RH), lambda i: (0, 0)),
            pl.BlockSpec((_RH, 2048), lambda i: (0, 0)),
            pl.BlockSpec((_RH, 1024), lambda i: (0, 0)),
            pl.BlockSpec((_RH, 1024), lambda i: (0, 0)),
            pl.BlockSpec((_RH, 1024), lambda i: (0, 0)),
        ],
        out_specs=pl.BlockSpec((_EBLK, _DIM), lambda i: (i, 0)),
        out_shape=jax.ShapeDtypeStruct((edges, _DIM), jnp.float32),
    )(attr2, hsrc, rad_w1, w_s, w_4, w_3, r_rep)


def _scatter_call(fdst2, msg, zeros_hbm):
    edges = fdst2.shape[0] * fdst2.shape[1]
    bn = zeros_hbm.shape[0]
    per_w = edges // (_NC * _NS)
    rows_per_w = fdst2.shape[1]
    k = per_w // rows_per_w
    rows_per_tile = bn // _NS
    mesh = plsc.VectorSubcoreMesh(core_axis_name="c", subcore_axis_name="s",
                                  num_cores=_NC, num_subcores=_NS)

    @functools.partial(
        pl.kernel,
        out_type=jax.ShapeDtypeStruct((_NC, bn, _DIM), jnp.float32),
        mesh=mesh,
        scratch_types=[
            pltpu.VMEM((k, rows_per_w), jnp.int32),
            pltpu.VMEM((per_w, _DIM), jnp.float32),
            pltpu.VMEM_SHARED((bn, _DIM), jnp.float32),
        ],
    )
    def _scatter(idx_hbm, msg_hbm, z_hbm, out_hbm, idx_v, rows_v, acc):
        cid = lax.axis_index("c")
        sid = lax.axis_index("s")
        wid = sid * _NC + cid
        r0 = sid * rows_per_tile
        pltpu.sync_copy(z_hbm.at[pl.ds(r0, rows_per_tile)],
                        acc.at[pl.ds(r0, rows_per_tile)])
        pltpu.sync_copy(idx_hbm.at[pl.ds(wid * k, k)], idx_v)
        pltpu.sync_copy(msg_hbm.at[pl.ds(wid * per_w, per_w)], rows_v)
        plsc.subcore_barrier()
        for j in range(k):
            pltpu.sync_copy(rows_v.at[pl.ds(j * rows_per_w, rows_per_w)],
                            acc.at[idx_v.at[j]], add=True)
        plsc.subcore_barrier()
        pltpu.sync_copy(acc.at[pl.ds(r0, rows_per_tile)],
                        out_hbm.at[cid, pl.ds(r0, rows_per_tile)])

    return _scatter(fdst2, msg, zeros_hbm)


def _fin_body(part_ref, h_ref, p_ref, ws_ref, bs_ref, wv_ref, pout_ref,
              out_ref):
    hp = jnp.dot(h_ref[...], p_ref[...], preferred_element_type=jnp.float32)
    res = _equiv_planar(hp, ws_ref[...], bs_ref[...], wv_ref[...])
    x = part_ref[0] + part_ref[1] + res
    s = x[:, 0:_MUL]
    v0 = x[:, _MUL:2 * _MUL]
    v1 = x[:, 2 * _MUL:3 * _MUL]
    v2 = x[:, 3 * _MUL:4 * _MUL]
    ns = jnp.abs(s)
    s2 = s * (ns * jax.nn.sigmoid(ns)) / (ns + 1e-8)
    nv = jnp.sqrt(v0 * v0 + v1 * v1 + v2 * v2)
    f = (nv * jax.nn.sigmoid(nv)) / (nv + 1e-8)
    y = jnp.concatenate([s2, v0 * f, v1 * f, v2 * f], axis=1)
    out_ref[...] = jnp.dot(y, pout_ref[...],
                           preferred_element_type=jnp.float32)


def _fin_call(parts, h2, p_in, res_ws, res_bs, res_wv, pout):
    bn = h2.shape[0]
    return pl.pallas_call(
        _fin_body,
        out_shape=jax.ShapeDtypeStruct((bn, _DIM), jnp.float32),
    )(parts, h2, p_in, res_ws, res_bs, res_wv, pout)


def kernel(h, e_src, e_dst, e_attr3, li_ws, li_bs, li_wv,
           rad_w1, rad_w2, res_ws, res_bs, res_wv):
    b, n, c = h.shape
    e = e_attr3.shape[1]
    edges = b * e

    p_in = jnp.asarray(_P_PLANAR)
    rad4 = rad_w2.reshape(_RH, 4, _MUL, _MUL)
    def perm(p):
        return rad4[:, p].transpose(1, 0, 2).reshape(_MUL, _RH * _MUL)
    w_s = jnp.concatenate([perm(0), perm(1)], axis=1)
    w_4 = perm(3)
    w_3 = perm(2)
    r_rep = jnp.asarray(np.repeat(np.eye(_RH, dtype=np.float32), _MUL, axis=1))
    pout = jnp.asarray(_P_OUT)
    zeros = jnp.zeros((b * n, _DIM), jnp.float32)

    h2 = h.reshape(b * n, c)
    hin, fsrc, fdst = _pre_call(h2, p_in, li_ws, li_bs.reshape(1, _MUL),
                                li_wv, e_src, e_dst)
    idx_cols = 128
    hsrc = _gather_call(fsrc.reshape(edges // idx_cols, idx_cols), hin)
    msg = _msg_call(e_attr3.reshape(edges, 3), hsrc,
                    rad_w1, w_s, w_4, w_3, r_rep)
    parts = _scatter_call(fdst.reshape(edges // idx_cols, idx_cols), msg, zeros)
    out = _fin_call(parts, h2, p_in, res_ws, res_bs.reshape(1, _MUL),
                    res_wv, pout)
    return out.reshape(b, n, c)


# speedup vs baseline: 3.7098x; 1.1358x over previous
import functools

import jax
import jax.numpy as jnp
import numpy as np
from jax import lax
from jax.experimental import pallas as pl
from jax.experimental.pallas import tpu as pltpu
from jax.experimental.pallas import tpu_sc as plsc

_MUL = 32
_DIM = 128
_RH = 32
_SQ3 = float(np.sqrt(3.0))
_SCALE = float(1.0 / np.sqrt(2.0 * _MUL) / np.sqrt(_RH))

_NC = 2
_NS = 16


def _perm_planar() -> np.ndarray:
    perm = np.zeros(_DIM, dtype=np.int64)
    perm[:_MUL] = np.arange(_MUL)
    for m in range(3):
        for i in range(_MUL):
            perm[_MUL + _MUL * m + i] = _MUL + 3 * i + m
    return np.eye(_DIM, dtype=np.float32)[perm].T


_P_PLANAR = _perm_planar()
_P_OUT = np.ascontiguousarray(_P_PLANAR.T)


def _equiv_planar(hp, ws, bs, wv):
    inv = float(1.0 / np.sqrt(_MUL))
    s = jnp.dot(hp[:, 0:_MUL], ws, preferred_element_type=jnp.float32) * inv
    s = s + bs
    outs = [s]
    for m in range(3):
        a = _MUL + _MUL * m
        outs.append(jnp.dot(hp[:, a:a + _MUL], wv,
                            preferred_element_type=jnp.float32) * inv)
    return jnp.concatenate(outs, axis=1)


def _pre_body(h_ref, p_ref, ws_ref, bs_ref, wv_ref, esrc_ref, edst_ref,
              hin_ref, fsrc_ref, fdst_ref):
    hp = jnp.dot(h_ref[...], p_ref[...], preferred_element_type=jnp.float32)
    hin_ref[...] = _equiv_planar(hp, ws_ref[...], bs_ref[...], wv_ref[...])
    n = hin_ref.shape[0] // esrc_ref.shape[0]
    boff = lax.broadcasted_iota(jnp.int32, esrc_ref.shape, 0) * n
    fsrc_ref[...] = esrc_ref[...] + boff
    fdst_ref[...] = edst_ref[...] + boff


def _pre_call(h2, p_in, li_ws, li_bs, li_wv, e_src, e_dst):
    bn = h2.shape[0]
    b, e = e_src.shape
    return pl.pallas_call(
        _pre_body,
        out_shape=[
            jax.ShapeDtypeStruct((bn, _DIM), jnp.float32),
            jax.ShapeDtypeStruct((b, e), jnp.int32),
            jax.ShapeDtypeStruct((b, e), jnp.int32),
        ],
    )(h2, p_in, li_ws, li_bs, li_wv, e_src, e_dst)


def _gather_call(fsrc2, hin):
    edges = fsrc2.shape[0] * fsrc2.shape[1]
    per_w = edges // (_NC * _NS)
    rows_per_w = fsrc2.shape[1]
    k = per_w // rows_per_w
    mesh = plsc.VectorSubcoreMesh(core_axis_name="c", subcore_axis_name="s",
                                  num_cores=_NC, num_subcores=_NS)

    @functools.partial(
        pl.kernel,
        out_type=jax.ShapeDtypeStruct((edges, _DIM), jnp.float32),
        mesh=mesh,
        scratch_types=[
            pltpu.VMEM((k, rows_per_w), jnp.int32),
            pltpu.VMEM((per_w, _DIM), jnp.float32),
            pltpu.SemaphoreType.DMA,
        ],
    )
    def _gather(idx_hbm, table_hbm, out_hbm, idx_v, rows_v, sem):
        wid = lax.axis_index("s") * _NC + lax.axis_index("c")
        pltpu.sync_copy(idx_hbm.at[pl.ds(wid * k, k)], idx_v)
        cps = [
            pltpu.async_copy(table_hbm.at[idx_v.at[j]],
                             rows_v.at[pl.ds(j * rows_per_w, rows_per_w)], sem)
            for j in range(k)
        ]
        for c in cps:
            c.wait()
        pltpu.sync_copy(rows_v, out_hbm.at[pl.ds(wid * per_w, per_w)])

    return _gather(fsrc2, hin)


_EBLK = 512


def _msg_body(attr_ref, hsrc_ref, rw1_ref, ws_ref, w4_ref, w3_ref, rrep_ref,
              out_ref):
    att = attr_ref[...]
    r0, r1, r2 = att[:, 0:1], att[:, 1:2], att[:, 2:3]
    rnorm = jnp.sqrt(r0 * r0 + r1 * r1 + r2 * r2) + 1e-8
    inv = 1.0 / rnorm
    y0, y1, y2 = _SQ3 * r0 * inv, _SQ3 * r1 * inv, _SQ3 * r2 * inv
    pre = rnorm * rw1_ref[...]
    hid = pre * jax.nn.sigmoid(pre)
    hs = hsrc_ref[...]
    s = hs[:, 0:_MUL]
    v0 = hs[:, _MUL:2 * _MUL]
    v1 = hs[:, 2 * _MUL:3 * _MUL]
    v2 = hs[:, 3 * _MUL:4 * _MUL]
    inner = (v0 * y0 + v1 * y1 + v2 * y2) * (1.0 / _SQ3)
    hrep = jnp.dot(hid, rrep_ref[...], preferred_element_type=jnp.float32)
    hlo, hhi = hrep[:, :512], hrep[:, 512:]
    bf = jnp.bfloat16

    def contract(t):
        x = hlo * t[:, :512] + hhi * t[:, 512:]
        x = x[:, :256] + x[:, 256:]
        x = x[:, :128] + x[:, 128:]
        x = x[:, :64] + x[:, 64:]
        return x[:, :32] + x[:, 32:]

    def bdot(a, w):
        return jnp.dot(a.astype(bf), w,
                       preferred_element_type=jnp.float32)

    ts = bdot(s, ws_ref[...])
    o1 = contract(ts[:, :1024])
    o2 = contract(ts[:, 1024:])
    o4 = contract(bdot(inner, w4_ref[...]))
    o30 = contract(bdot(v0, w3_ref[...]))
    o31 = contract(bdot(v1, w3_ref[...]))
    o32 = contract(bdot(v2, w3_ref[...]))
    m_s = _SCALE * (o1 + o4)
    out_ref[...] = jnp.concatenate(
        [m_s,
         _SCALE * (o2 * y0 + o30),
         _SCALE * (o2 * y1 + o31),
         _SCALE * (o2 * y2 + o32)], axis=1)


def _msg_call(attr2, hsrc, rad_w1, w_s, w_4, w_3, r_rep):
    edges = attr2.shape[0]
    grid = edges // _EBLK
    return pl.pallas_call(
        _msg_body,
        grid=(grid,),
        in_specs=[
            pl.BlockSpec((_EBLK, 3), lambda i: (i, 0)),
            pl.BlockSpec((_EBLK, _DIM), lambda i: (i, 0)),
            pl.BlockSpec((1, _RH), lambda i: (0, 0)),
            pl.BlockSpec((_RH, 2048), lambda i: (0, 0)),
            pl.BlockSpec((_RH, 1024), lambda i: (0, 0)),
            pl.BlockSpec((_RH, 1024), lambda i: (0, 0)),
            pl.BlockSpec((_RH, 1024), lambda i: (0, 0)),
        ],
        out_specs=pl.BlockSpec((_EBLK, _DIM), lambda i: (i, 0)),
        out_shape=jax.ShapeDtypeStruct((edges, _DIM), jnp.float32),
    )(attr2, hsrc, rad_w1, w_s, w_4, w_3, r_rep)


def _scatter_call(fdst2, msg, zeros_hbm):
    edges = fdst2.shape[0] * fdst2.shape[1]
    bn = zeros_hbm.shape[0]
    per_w = edges // (_NC * _NS)
    rows_per_w = fdst2.shape[1]
    k = per_w // rows_per_w
    rows_per_tile = bn // _NS
    mesh = plsc.VectorSubcoreMesh(core_axis_name="c", subcore_axis_name="s",
                                  num_cores=_NC, num_subcores=_NS)

    @functools.partial(
        pl.kernel,
        out_type=jax.ShapeDtypeStruct((_NC, bn, _DIM), jnp.float32),
        mesh=mesh,
        scratch_types=[
            pltpu.VMEM((k, rows_per_w), jnp.int32),
            pltpu.VMEM((per_w, _DIM), jnp.float32),
            pltpu.VMEM_SHARED((bn, _DIM), jnp.float32),
        ],
    )
    def _scatter(idx_hbm, msg_hbm, z_hbm, out_hbm, idx_v, rows_v, acc):
        cid = lax.axis_index("c")
        sid = lax.axis_index("s")
        wid = sid * _NC + cid
        r0 = sid * rows_per_tile
        pltpu.sync_copy(z_hbm.at[pl.ds(r0, rows_per_tile)],
                        acc.at[pl.ds(r0, rows_per_tile)])
        pltpu.sync_copy(idx_hbm.at[pl.ds(wid * k, k)], idx_v)
        pltpu.sync_copy(msg_hbm.at[pl.ds(wid * per_w, per_w)], rows_v)
        plsc.subcore_barrier()
        for j in range(k):
            pltpu.sync_copy(rows_v.at[pl.ds(j * rows_per_w, rows_per_w)],
                            acc.at[idx_v.at[j]], add=True)
        plsc.subcore_barrier()
        pltpu.sync_copy(acc.at[pl.ds(r0, rows_per_tile)],
                        out_hbm.at[cid, pl.ds(r0, rows_per_tile)])

    return _scatter(fdst2, msg, zeros_hbm)


def _fin_body(part_ref, h_ref, p_ref, ws_ref, bs_ref, wv_ref, pout_ref,
              out_ref):
    hp = jnp.dot(h_ref[...], p_ref[...], preferred_element_type=jnp.float32)
    res = _equiv_planar(hp, ws_ref[...], bs_ref[...], wv_ref[...])
    x = part_ref[0] + part_ref[1] + res
    s = x[:, 0:_MUL]
    v0 = x[:, _MUL:2 * _MUL]
    v1 = x[:, 2 * _MUL:3 * _MUL]
    v2 = x[:, 3 * _MUL:4 * _MUL]
    ns = jnp.abs(s)
    s2 = s * (ns * jax.nn.sigmoid(ns)) / (ns + 1e-8)
    nv = jnp.sqrt(v0 * v0 + v1 * v1 + v2 * v2)
    f = (nv * jax.nn.sigmoid(nv)) / (nv + 1e-8)
    y = jnp.concatenate([s2, v0 * f, v1 * f, v2 * f], axis=1)
    out_ref[...] = jnp.dot(y, pout_ref[...],
                           preferred_element_type=jnp.float32)


def _fin_call(parts, h2, p_in, res_ws, res_bs, res_wv, pout):
    bn = h2.shape[0]
    return pl.pallas_call(
        _fin_body,
        out_shape=jax.ShapeDtypeStruct((bn, _DIM), jnp.float32),
    )(parts, h2, p_in, res_ws, res_bs, res_wv, pout)


def kernel(h, e_src, e_dst, e_attr3, li_ws, li_bs, li_wv,
           rad_w1, rad_w2, res_ws, res_bs, res_wv):
    b, n, c = h.shape
    e = e_attr3.shape[1]
    edges = b * e

    p_in = jnp.asarray(_P_PLANAR)
    rad4 = rad_w2.reshape(_RH, 4, _MUL, _MUL)
    def perm(p):
        return rad4[:, p].transpose(1, 0, 2).reshape(_MUL, _RH * _MUL)
    w_s = jnp.concatenate([perm(0), perm(1)], axis=1).astype(jnp.bfloat16)
    w_4 = perm(3).astype(jnp.bfloat16)
    w_3 = perm(2).astype(jnp.bfloat16)
    r_rep = jnp.asarray(np.repeat(np.eye(_RH, dtype=np.float32), _MUL, axis=1))
    pout = jnp.asarray(_P_OUT)
    zeros = jnp.zeros((b * n, _DIM), jnp.float32)

    h2 = h.reshape(b * n, c)
    hin, fsrc, fdst = _pre_call(h2, p_in, li_ws, li_bs.reshape(1, _MUL),
                                li_wv, e_src, e_dst)
    idx_cols = 128
    hsrc = _gather_call(fsrc.reshape(edges // idx_cols, idx_cols), hin)
    msg = _msg_call(e_attr3.reshape(edges, 3), hsrc,
                    rad_w1, w_s, w_4, w_3, r_rep)
    parts = _scatter_call(fdst.reshape(edges // idx_cols, idx_cols), msg, zeros)
    out = _fin_call(parts, h2, p_in, res_ws, res_bs.reshape(1, _MUL),
                    res_wv, pout)
    return out.reshape(b, n, c)


# speedup vs baseline: 3.9729x; 1.0709x over previous
import functools

import jax
import jax.numpy as jnp
import numpy as np
from jax import lax
from jax.experimental import pallas as pl
from jax.experimental.pallas import tpu as pltpu
from jax.experimental.pallas import tpu_sc as plsc

_MUL = 32
_DIM = 128
_RH = 32
_SQ3 = float(np.sqrt(3.0))
_SCALE = float(1.0 / np.sqrt(2.0 * _MUL) / np.sqrt(_RH))

_NC = 2
_NS = 16


def _perm_planar() -> np.ndarray:
    perm = np.zeros(_DIM, dtype=np.int64)
    perm[:_MUL] = np.arange(_MUL)
    for m in range(3):
        for i in range(_MUL):
            perm[_MUL + _MUL * m + i] = _MUL + 3 * i + m
    return np.eye(_DIM, dtype=np.float32)[perm].T


_P_PLANAR = _perm_planar()
_P_OUT = np.ascontiguousarray(_P_PLANAR.T)


def _equiv_planar(hp, ws, bs, wv):
    inv = float(1.0 / np.sqrt(_MUL))
    s = jnp.dot(hp[:, 0:_MUL], ws, preferred_element_type=jnp.float32) * inv
    s = s + bs
    outs = [s]
    for m in range(3):
        a = _MUL + _MUL * m
        outs.append(jnp.dot(hp[:, a:a + _MUL], wv,
                            preferred_element_type=jnp.float32) * inv)
    return jnp.concatenate(outs, axis=1)


def _pre_body(h_ref, p_ref, ws_ref, bs_ref, wv_ref, esrc_ref, edst_ref,
              hin_ref, fsrc_ref, fdst_ref):
    hp = jnp.dot(h_ref[...], p_ref[...], preferred_element_type=jnp.float32)
    hin_ref[...] = _equiv_planar(hp, ws_ref[...], bs_ref[...], wv_ref[...])
    n = hin_ref.shape[0] // esrc_ref.shape[0]
    boff = lax.broadcasted_iota(jnp.int32, esrc_ref.shape, 0) * n
    fsrc_ref[...] = esrc_ref[...] + boff
    fdst_ref[...] = edst_ref[...] + boff


def _pre_call(h2, p_in, li_ws, li_bs, li_wv, e_src, e_dst):
    bn = h2.shape[0]
    b, e = e_src.shape
    return pl.pallas_call(
        _pre_body,
        out_shape=[
            jax.ShapeDtypeStruct((bn, _DIM), jnp.float32),
            jax.ShapeDtypeStruct((b, e), jnp.int32),
            jax.ShapeDtypeStruct((b, e), jnp.int32),
        ],
    )(h2, p_in, li_ws, li_bs, li_wv, e_src, e_dst)


def _gather_call(fsrc2, hin):
    edges = fsrc2.shape[0] * fsrc2.shape[1]
    per_w = edges // (_NC * _NS)
    rows_per_w = fsrc2.shape[1]
    k = per_w // rows_per_w
    mesh = plsc.VectorSubcoreMesh(core_axis_name="c", subcore_axis_name="s",
                                  num_cores=_NC, num_subcores=_NS)

    @functools.partial(
        pl.kernel,
        out_type=jax.ShapeDtypeStruct((edges, _DIM), jnp.float32),
        mesh=mesh,
        scratch_types=[
            pltpu.VMEM((k, rows_per_w), jnp.int32),
            pltpu.VMEM((per_w, _DIM), jnp.float32),
            pltpu.SemaphoreType.DMA,
        ],
    )
    def _gather(idx_hbm, table_hbm, out_hbm, idx_v, rows_v, sem):
        wid = lax.axis_index("s") * _NC + lax.axis_index("c")
        pltpu.sync_copy(idx_hbm.at[pl.ds(wid * k, k)], idx_v)
        cps = [
            pltpu.async_copy(table_hbm.at[idx_v.at[j]],
                             rows_v.at[pl.ds(j * rows_per_w, rows_per_w)], sem)
            for j in range(k)
        ]
        for c in cps:
            c.wait()
        pltpu.sync_copy(rows_v, out_hbm.at[pl.ds(wid * per_w, per_w)])

    return _gather(fsrc2, hin)


_EBLK = 1024


def _msg_body(attr_ref, hsrc_ref, rw1_ref, ws_ref, w4_ref, w3_ref, rrep_ref,
              out_ref):
    att = attr_ref[...]
    r0, r1, r2 = att[:, 0:1], att[:, 1:2], att[:, 2:3]
    rnorm = jnp.sqrt(r0 * r0 + r1 * r1 + r2 * r2) + 1e-8
    inv = 1.0 / rnorm
    y0, y1, y2 = _SQ3 * r0 * inv, _SQ3 * r1 * inv, _SQ3 * r2 * inv
    pre = rnorm * rw1_ref[...]
    hid = pre * jax.nn.sigmoid(pre)
    hs = hsrc_ref[...]
    s = hs[:, 0:_MUL]
    v0 = hs[:, _MUL:2 * _MUL]
    v1 = hs[:, 2 * _MUL:3 * _MUL]
    v2 = hs[:, 3 * _MUL:4 * _MUL]
    inner = (v0 * y0 + v1 * y1 + v2 * y2) * (1.0 / _SQ3)
    hrep = jnp.dot(hid, rrep_ref[...], preferred_element_type=jnp.float32)
    hlo, hhi = hrep[:, :512], hrep[:, 512:]
    bf = jnp.bfloat16

    def contract(t):
        x = hlo * t[:, :512] + hhi * t[:, 512:]
        x = x[:, :256] + x[:, 256:]
        x = x[:, :128] + x[:, 128:]
        x = x[:, :64] + x[:, 64:]
        return x[:, :32] + x[:, 32:]

    def bdot(a, w):
        return jnp.dot(a.astype(bf), w,
                       preferred_element_type=jnp.float32)

    ts = bdot(s, ws_ref[...])
    o1 = contract(ts[:, :1024])
    o2 = contract(ts[:, 1024:])
    o4 = contract(bdot(inner, w4_ref[...]))
    o30 = contract(bdot(v0, w3_ref[...]))
    o31 = contract(bdot(v1, w3_ref[...]))
    o32 = contract(bdot(v2, w3_ref[...]))
    m_s = _SCALE * (o1 + o4)
    out_ref[...] = jnp.concatenate(
        [m_s,
         _SCALE * (o2 * y0 + o30),
         _SCALE * (o2 * y1 + o31),
         _SCALE * (o2 * y2 + o32)], axis=1)


def _msg_call(attr2, hsrc, rad_w1, w_s, w_4, w_3, r_rep):
    edges = attr2.shape[0]
    grid = edges // _EBLK
    return pl.pallas_call(
        _msg_body,
        grid=(grid,),
        in_specs=[
            pl.BlockSpec((_EBLK, 3), lambda i: (i, 0)),
            pl.BlockSpec((_EBLK, _DIM), lambda i: (i, 0)),
            pl.BlockSpec((1, _RH), lambda i: (0, 0)),
            pl.BlockSpec((_RH, 2048), lambda i: (0, 0)),
            pl.BlockSpec((_RH, 1024), lambda i: (0, 0)),
            pl.BlockSpec((_RH, 1024), lambda i: (0, 0)),
            pl.BlockSpec((_RH, 1024), lambda i: (0, 0)),
        ],
        out_specs=pl.BlockSpec((_EBLK, _DIM), lambda i: (i, 0)),
        out_shape=jax.ShapeDtypeStruct((edges, _DIM), jnp.float32),
    )(attr2, hsrc, rad_w1, w_s, w_4, w_3, r_rep)


def _scatter_call(fdst2, msg, zeros_hbm):
    edges = fdst2.shape[0] * fdst2.shape[1]
    bn = zeros_hbm.shape[0]
    per_w = edges // (_NC * _NS)
    rows_per_w = fdst2.shape[1]
    k = per_w // rows_per_w
    rows_per_tile = bn // _NS
    mesh = plsc.VectorSubcoreMesh(core_axis_name="c", subcore_axis_name="s",
                                  num_cores=_NC, num_subcores=_NS)

    @functools.partial(
        pl.kernel,
        out_type=jax.ShapeDtypeStruct((_NC, bn, _DIM), jnp.float32),
        mesh=mesh,
        scratch_types=[
            pltpu.VMEM((k, rows_per_w), jnp.int32),
            pltpu.VMEM((per_w, _DIM), jnp.float32),
            pltpu.VMEM_SHARED((bn, _DIM), jnp.float32),
        ],
    )
    def _scatter(idx_hbm, msg_hbm, z_hbm, out_hbm, idx_v, rows_v, acc):
        cid = lax.axis_index("c")
        sid = lax.axis_index("s")
        wid = sid * _NC + cid
        r0 = sid * rows_per_tile
        pltpu.sync_copy(z_hbm.at[pl.ds(r0, rows_per_tile)],
                        acc.at[pl.ds(r0, rows_per_tile)])
        pltpu.sync_copy(idx_hbm.at[pl.ds(wid * k, k)], idx_v)
        pltpu.sync_copy(msg_hbm.at[pl.ds(wid * per_w, per_w)], rows_v)
        plsc.subcore_barrier()
        for j in range(k):
            pltpu.sync_copy(rows_v.at[pl.ds(j * rows_per_w, rows_per_w)],
                            acc.at[idx_v.at[j]], add=True)
        plsc.subcore_barrier()
        pltpu.sync_copy(acc.at[pl.ds(r0, rows_per_tile)],
                        out_hbm.at[cid, pl.ds(r0, rows_per_tile)])

    return _scatter(fdst2, msg, zeros_hbm)


def _fin_body(part_ref, h_ref, p_ref, ws_ref, bs_ref, wv_ref, pout_ref,
              out_ref):
    hp = jnp.dot(h_ref[...], p_ref[...], preferred_element_type=jnp.float32)
    res = _equiv_planar(hp, ws_ref[...], bs_ref[...], wv_ref[...])
    x = part_ref[0] + part_ref[1] + res
    s = x[:, 0:_MUL]
    v0 = x[:, _MUL:2 * _MUL]
    v1 = x[:, 2 * _MUL:3 * _MUL]
    v2 = x[:, 3 * _MUL:4 * _MUL]
    ns = jnp.abs(s)
    s2 = s * (ns * jax.nn.sigmoid(ns)) / (ns + 1e-8)
    nv = jnp.sqrt(v0 * v0 + v1 * v1 + v2 * v2)
    f = (nv * jax.nn.sigmoid(nv)) / (nv + 1e-8)
    y = jnp.concatenate([s2, v0 * f, v1 * f, v2 * f], axis=1)
    out_ref[...] = jnp.dot(y, pout_ref[...],
                           preferred_element_type=jnp.float32)


def _fin_call(parts, h2, p_in, res_ws, res_bs, res_wv, pout):
    bn = h2.shape[0]
    return pl.pallas_call(
        _fin_body,
        out_shape=jax.ShapeDtypeStruct((bn, _DIM), jnp.float32),
    )(parts, h2, p_in, res_ws, res_bs, res_wv, pout)


def kernel(h, e_src, e_dst, e_attr3, li_ws, li_bs, li_wv,
           rad_w1, rad_w2, res_ws, res_bs, res_wv):
    b, n, c = h.shape
    e = e_attr3.shape[1]
    edges = b * e

    p_in = jnp.asarray(_P_PLANAR)
    rad4 = rad_w2.reshape(_RH, 4, _MUL, _MUL)
    def perm(p):
        return rad4[:, p].transpose(1, 0, 2).reshape(_MUL, _RH * _MUL)
    w_s = jnp.concatenate([perm(0), perm(1)], axis=1).astype(jnp.bfloat16)
    w_4 = perm(3).astype(jnp.bfloat16)
    w_3 = perm(2).astype(jnp.bfloat16)
    r_rep = jnp.asarray(np.repeat(np.eye(_RH, dtype=np.float32), _MUL, axis=1))
    pout = jnp.asarray(_P_OUT)
    zeros = jnp.zeros((b * n, _DIM), jnp.float32)

    h2 = h.reshape(b * n, c)
    hin, fsrc, fdst = _pre_call(h2, p_in, li_ws, li_bs.reshape(1, _MUL),
                                li_wv, e_src, e_dst)
    idx_cols = 128
    hsrc = _gather_call(fsrc.reshape(edges // idx_cols, idx_cols), hin)
    msg = _msg_call(e_attr3.reshape(edges, 3), hsrc,
                    rad_w1, w_s, w_4, w_3, r_rep)
    parts = _scatter_call(fdst.reshape(edges // idx_cols, idx_cols), msg, zeros)
    out = _fin_call(parts, h2, p_in, res_ws, res_bs.reshape(1, _MUL),
                    res_wv, pout)
    return out.reshape(b, n, c)
